# Initial kernel scaffold; baseline (speedup 1.0000x reference)
import jax
import jax.numpy as jnp
from jax.experimental import pallas as pl


def kernel(x, edge_index, ln_gamma, ln_beta, W1, b1, W2, b2):
    raise NotImplementedError("write your pallas kernel here")


# speedup vs baseline: 19.0395x; 19.0395x over previous
import functools

import jax
import jax.numpy as jnp
from jax import lax
from jax.experimental import pallas as pl
from jax.experimental.pallas import tpu as pltpu
from jax.experimental.pallas import tpu_sc as plsc

N = 10000
E = 320000
D_IN = 128
D = 16

NC = 2
NS = 16
NW = NC * NS

NP = 10240
ROWS_PER_TILE = NP // NS

CHUNK = 128
EDGES_PER_TILE = 10112
EP = EDGES_PER_TILE * NW
NCHUNK = EDGES_PER_TILE // CHUNK

_SC_MESH = plsc.VectorSubcoreMesh(
    core_axis_name="c", subcore_axis_name="s", num_cores=NC, num_subcores=NS
)


def _tile_ids():
    cid = lax.axis_index("c")
    sid = lax.axis_index("s")
    return cid, sid


@functools.partial(
    pl.kernel,
    out_type=jax.ShapeDtypeStruct((NC * NP,), jnp.float32),
    mesh=_SC_MESH,
    scratch_types=[
        pltpu.VMEM((CHUNK,), jnp.float32),
        pltpu.VMEM((CHUNK,), jnp.int32),
        pltpu.VMEM_SHARED((NP,), jnp.float32),
    ],
)
def _deg_kernel(dst_hbm, out_hbm, ones_v, idx_v, deg_sh):
    cid, sid = _tile_ids()
    edge_base = (cid * NS + sid) * EDGES_PER_TILE

    def fill(i, _):
        ones_v[pl.ds(i * 16, 16)] = jnp.ones((16,), jnp.float32)
        return ()
    lax.fori_loop(0, CHUNK // 16, fill, ())

    def zfill(i, _):
        ones_v[pl.ds(i * 16, 16)] = jnp.zeros((16,), jnp.float32)
        return ()
    lax.fori_loop(0, CHUNK // 16, zfill, ())
    for k in range(ROWS_PER_TILE // CHUNK):
        pltpu.sync_copy(ones_v, deg_sh.at[pl.ds(sid * ROWS_PER_TILE + k * CHUNK, CHUNK)])
    lax.fori_loop(0, CHUNK // 16, fill, ())
    plsc.subcore_barrier()

    def chunk_body(c, _):
        base = edge_base + c * CHUNK
        pltpu.sync_copy(dst_hbm.at[pl.ds(base, CHUNK)], idx_v)
        pltpu.sync_copy(ones_v, deg_sh.at[idx_v], add=True)
        return ()
    lax.fori_loop(0, NCHUNK, chunk_body, ())
    plsc.subcore_barrier()

    pltpu.sync_copy(
        deg_sh.at[pl.ds(sid * ROWS_PER_TILE, ROWS_PER_TILE)],
        out_hbm.at[pl.ds(cid * NP + sid * ROWS_PER_TILE, ROWS_PER_TILE)],
    )


@functools.partial(
    pl.kernel,
    out_type=jax.ShapeDtypeStruct((NC * NP, D), jnp.float32),
    mesh=_SC_MESH,
    scratch_types=[
        pltpu.VMEM((CHUNK,), jnp.int32),
        pltpu.VMEM((CHUNK,), jnp.int32),
        pltpu.VMEM((CHUNK, D), jnp.float32),
        pltpu.VMEM_SHARED((NP, D), jnp.float32),
        pltpu.SemaphoreType.DMA,
    ],
    compiler_params=pltpu.CompilerParams(use_tc_tiling_on_sc=False),
)
def _agg_kernel(table_hbm, src_hbm, dst_hbm, out_hbm, idx_s, idx_d, rows_v, acc_sh, sem):
    cid, sid = _tile_ids()
    edge_base = (cid * NS + sid) * EDGES_PER_TILE

    def zfill(i, _):
        rows_v[i, :] = jnp.zeros((D,), jnp.float32)
        return ()
    lax.fori_loop(0, CHUNK, zfill, ())
    for k in range(ROWS_PER_TILE // CHUNK):
        pltpu.sync_copy(rows_v, acc_sh.at[pl.ds(sid * ROWS_PER_TILE + k * CHUNK, CHUNK)])
    plsc.subcore_barrier()

    def chunk_body(c, _):
        base = edge_base + c * CHUNK
        pltpu.sync_copy(src_hbm.at[pl.ds(base, CHUNK)], idx_s)
        pltpu.async_copy(table_hbm.at[idx_s], rows_v, sem).wait()
        pltpu.sync_copy(dst_hbm.at[pl.ds(base, CHUNK)], idx_d)
        pltpu.sync_copy(rows_v, acc_sh.at[idx_d], add=True)
        return ()
    lax.fori_loop(0, NCHUNK, chunk_body, ())
    plsc.subcore_barrier()

    pltpu.sync_copy(
        acc_sh.at[pl.ds(sid * ROWS_PER_TILE, ROWS_PER_TILE)],
        out_hbm.at[pl.ds(cid * NP + sid * ROWS_PER_TILE, ROWS_PER_TILE)],
    )


_BR = 1024


def _tc1_body(x_ref, g_ref, b_ref, w_ref, degp_ref, hs_ref, dis_ref):
    xb = x_ref[...]
    mean = jnp.mean(xb, axis=1, keepdims=True)
    xc = xb - mean
    var = jnp.mean(xc * xc, axis=1, keepdims=True)
    xn = xc * lax.rsqrt(var + 1e-5) * g_ref[...] + b_ref[...]
    deg = degp_ref[0, :] + degp_ref[1, :] + 1.0
    dis = lax.rsqrt(deg)
    h = jnp.dot(xn, w_ref[...], preferred_element_type=jnp.float32)
    hs_ref[...] = h * dis[:, None]
    dis_ref[...] = dis[None, :]


def _tc1(xp, g2, b2d, W1, degp):
    return pl.pallas_call(
        _tc1_body,
        grid=(NP // _BR,),
        in_specs=[
            pl.BlockSpec((_BR, D_IN), lambda i: (i, 0)),
            pl.BlockSpec((1, D_IN), lambda i: (0, 0)),
            pl.BlockSpec((1, D_IN), lambda i: (0, 0)),
            pl.BlockSpec((D_IN, D), lambda i: (0, 0)),
            pl.BlockSpec((NC, _BR), lambda i: (0, i)),
        ],
        out_specs=[
            pl.BlockSpec((_BR, D), lambda i: (i, 0)),
            pl.BlockSpec((1, _BR), lambda i: (0, i)),
        ],
        out_shape=[
            jax.ShapeDtypeStruct((NP, D), jnp.float32),
            jax.ShapeDtypeStruct((1, NP), jnp.float32),
        ],
    )(xp, g2, b2d, W1, degp)


def _tc2_body(a0_ref, a1_ref, hs_ref, dis_ref, bias_ref, w_ref, out_ref):
    dis = dis_ref[0, :][:, None]
    t = (a0_ref[...] + a1_ref[...] + hs_ref[...]) * dis + bias_ref[...]
    t = jnp.maximum(t, 0.0)
    h2 = jnp.dot(t, w_ref[...], preferred_element_type=jnp.float32)
    out_ref[...] = h2 * dis


def _tc2(a0, a1, hs1, dis, bias, W2):
    return pl.pallas_call(
        _tc2_body,
        grid=(NP // _BR,),
        in_specs=[
            pl.BlockSpec((_BR, D), lambda i: (i, 0)),
            pl.BlockSpec((_BR, D), lambda i: (i, 0)),
            pl.BlockSpec((_BR, D), lambda i: (i, 0)),
            pl.BlockSpec((1, _BR), lambda i: (0, i)),
            pl.BlockSpec((1, D), lambda i: (0, 0)),
            pl.BlockSpec((D, D), lambda i: (0, 0)),
        ],
        out_specs=pl.BlockSpec((_BR, D), lambda i: (i, 0)),
        out_shape=jax.ShapeDtypeStruct((NP, D), jnp.float32),
    )(a0, a1, hs1, dis, bias, W2)


def _tc3_body(a0_ref, a1_ref, hs_ref, dis_ref, bias_ref, out_ref):
    dis = dis_ref[0, :][:, None]
    t = (a0_ref[...] + a1_ref[...] + hs_ref[...]) * dis + bias_ref[...]
    out_ref[...] = jnp.maximum(t, 0.0)


def _tc3(a0, a1, hs2, dis, bias):
    return pl.pallas_call(
        _tc3_body,
        grid=(NP // _BR,),
        in_specs=[
            pl.BlockSpec((_BR, D), lambda i: (i, 0)),
            pl.BlockSpec((_BR, D), lambda i: (i, 0)),
            pl.BlockSpec((_BR, D), lambda i: (i, 0)),
            pl.BlockSpec((1, _BR), lambda i: (0, i)),
            pl.BlockSpec((1, D), lambda i: (0, 0)),
        ],
        out_specs=pl.BlockSpec((_BR, D), lambda i: (i, 0)),
        out_shape=jax.ShapeDtypeStruct((NP, D), jnp.float32),
    )(a0, a1, hs2, dis, bias)


def kernel(x, edge_index, ln_gamma, ln_beta, W1, b1, W2, b2):
    src = edge_index[0].astype(jnp.int32)
    dst = edge_index[1].astype(jnp.int32)
    pad_idx = jnp.full((EP - E,), N, jnp.int32)
    srcp = jnp.concatenate([src, pad_idx])
    dstp = jnp.concatenate([dst, pad_idx])
    xp = jnp.pad(x, ((0, NP - N), (0, 0)))
    g2 = ln_gamma.reshape(1, D_IN)
    be2 = ln_beta.reshape(1, D_IN)
    b1r = b1.reshape(1, D)
    b2r = b2.reshape(1, D)

    degp = _deg_kernel(dstp).reshape(NC, NP)
    hs1, dis = _tc1(xp, g2, be2, W1, degp)
    a1 = _agg_kernel(hs1, srcp, dstp)
    hs2 = _tc2(a1[:NP], a1[NP:], hs1, dis, b1r, W2)
    a2 = _agg_kernel(hs2, srcp, dstp)
    out = _tc3(a2[:NP], a2[NP:], hs2, dis, b2r)
    return out[:N]


# speedup vs baseline: 37.7178x; 1.9810x over previous
import functools

import jax
import jax.numpy as jnp
from jax import lax
from jax.experimental import pallas as pl
from jax.experimental.pallas import tpu as pltpu
from jax.experimental.pallas import tpu_sc as plsc

N = 10000
E = 320000
D_IN = 128
D = 16

NC = 2
NS = 16
NW = NC * NS

NP = 10240
ROWS_PER_TILE = NP // NS

CHUNK = 128
EDGES_PER_TILE = 10240
EP = EDGES_PER_TILE * NW
NCHUNK = EDGES_PER_TILE // CHUNK
K = 8
NWAVES = NCHUNK // K

_SC_MESH = plsc.VectorSubcoreMesh(
    core_axis_name="c", subcore_axis_name="s", num_cores=NC, num_subcores=NS
)


def _tile_ids():
    cid = lax.axis_index("c")
    sid = lax.axis_index("s")
    return cid, sid


_DEG_K = 16


@functools.partial(
    pl.kernel,
    out_type=jax.ShapeDtypeStruct((NC * NP,), jnp.float32),
    mesh=_SC_MESH,
    scratch_types=[
        pltpu.VMEM((CHUNK,), jnp.float32),
        pltpu.VMEM((NCHUNK, CHUNK), jnp.int32),
        pltpu.VMEM_SHARED((NP,), jnp.float32),
        pltpu.SemaphoreType.DMA,
    ],
)
def _deg_kernel(dst_hbm, out_hbm, ones_v, idx_all, deg_sh, ssem):
    cid, sid = _tile_ids()
    tile = cid * NS + sid
    pltpu.sync_copy(dst_hbm.at[pl.ds(tile * NCHUNK, NCHUNK)], idx_all)

    def zfill(i, _):
        ones_v[pl.ds(i * 16, 16)] = jnp.zeros((16,), jnp.float32)
        return ()
    lax.fori_loop(0, CHUNK // 16, zfill, ())
    for k in range(ROWS_PER_TILE // CHUNK):
        pltpu.sync_copy(ones_v, deg_sh.at[pl.ds(sid * ROWS_PER_TILE + k * CHUNK, CHUNK)])

    def fill(i, _):
        ones_v[pl.ds(i * 16, 16)] = jnp.ones((16,), jnp.float32)
        return ()
    lax.fori_loop(0, CHUNK // 16, fill, ())
    plsc.subcore_barrier()

    nwaves = NCHUNK // _DEG_K

    def wave(w, _):
        @pl.when(w < nwaves)
        def _fire():
            for b in range(_DEG_K):
                pltpu.async_copy(ones_v, deg_sh.at[idx_all.at[w * _DEG_K + b]],
                                 ssem, add=True)

        @pl.when(w > 0)
        def _drain():
            for b in range(_DEG_K):
                pltpu.make_async_copy(ones_v, deg_sh.at[pl.ds(0, CHUNK)], ssem).wait()
        return ()
    lax.fori_loop(0, nwaves + 1, wave, ())
    plsc.subcore_barrier()

    pltpu.sync_copy(
        deg_sh.at[pl.ds(sid * ROWS_PER_TILE, ROWS_PER_TILE)],
        out_hbm.at[pl.ds(cid * NP + sid * ROWS_PER_TILE, ROWS_PER_TILE)],
    )


@functools.partial(
    pl.kernel,
    out_type=jax.ShapeDtypeStruct((NC * NP, D), jnp.float32),
    mesh=_SC_MESH,
    scratch_types=[
        pltpu.VMEM((NCHUNK, CHUNK), jnp.int32),
        pltpu.VMEM((NCHUNK, CHUNK), jnp.int32),
        pltpu.VMEM((2, K, CHUNK, D), jnp.float32),
        pltpu.VMEM_SHARED((NP, D), jnp.float32),
        pltpu.SemaphoreType.DMA,
        pltpu.SemaphoreType.DMA,
    ],
    compiler_params=pltpu.CompilerParams(use_tc_tiling_on_sc=False),
)
def _agg_kernel(table_hbm, src_hbm, dst_hbm, out_hbm,
                idx_s, idx_d, rows_v, acc_sh, gsem, ssem):
    cid, sid = _tile_ids()
    tile = cid * NS + sid
    pltpu.sync_copy(src_hbm.at[pl.ds(tile * NCHUNK, NCHUNK)], idx_s)
    pltpu.sync_copy(dst_hbm.at[pl.ds(tile * NCHUNK, NCHUNK)], idx_d)

    def zfill(i, _):
        rows_v[0, 0, i, :] = jnp.zeros((D,), jnp.float32)
        return ()
    lax.fori_loop(0, CHUNK, zfill, ())
    for k in range(ROWS_PER_TILE // CHUNK):
        pltpu.sync_copy(rows_v.at[0, 0],
                        acc_sh.at[pl.ds(sid * ROWS_PER_TILE + k * CHUNK, CHUNK)])
    plsc.subcore_barrier()

    for b in range(K):
        pltpu.async_copy(table_hbm.at[idx_s.at[b]], rows_v.at[0, b], gsem)

    def wave(w, _):
        p = lax.rem(w, 2)
        c0 = w * K
        for b in range(K):
            pltpu.make_async_copy(table_hbm.at[pl.ds(0, CHUNK)],
                                  rows_v.at[p, b], gsem).wait()
        for b in range(K):
            pltpu.async_copy(rows_v.at[p, b], acc_sh.at[idx_d.at[c0 + b]],
                             ssem, add=True)

        @pl.when(w + 1 < NWAVES)
        def _fire_next():
            for b in range(K):
                pltpu.async_copy(table_hbm.at[idx_s.at[c0 + K + b]],
                                 rows_v.at[1 - p, b], gsem)

        for b in range(K):
            pltpu.make_async_copy(rows_v.at[p, b],
                                  acc_sh.at[pl.ds(0, CHUNK)], ssem).wait()
        return ()
    lax.fori_loop(0, NWAVES, wave, ())
    plsc.subcore_barrier()

    pltpu.sync_copy(
        acc_sh.at[pl.ds(sid * ROWS_PER_TILE, ROWS_PER_TILE)],
        out_hbm.at[pl.ds(cid * NP + sid * ROWS_PER_TILE, ROWS_PER_TILE)],
    )


_BR = 1024


def _tc1_body(x_ref, g_ref, b_ref, w_ref, degp_ref, hs_ref, dis_ref):
    xb = x_ref[...]
    mean = jnp.mean(xb, axis=1, keepdims=True)
    xc = xb - mean
    var = jnp.mean(xc * xc, axis=1, keepdims=True)
    xn = xc * lax.rsqrt(var + 1e-5) * g_ref[...] + b_ref[...]
    deg = degp_ref[0, :] + degp_ref[1, :] + 1.0
    dis = lax.rsqrt(deg)
    h = jnp.dot(xn, w_ref[...], preferred_element_type=jnp.float32)
    hs_ref[...] = h * dis[:, None]
    dis_ref[...] = dis[None, :]


def _tc1(xp, g2, b2d, W1, degp):
    return pl.pallas_call(
        _tc1_body,
        grid=(NP // _BR,),
        in_specs=[
            pl.BlockSpec((_BR, D_IN), lambda i: (i, 0)),
            pl.BlockSpec((1, D_IN), lambda i: (0, 0)),
            pl.BlockSpec((1, D_IN), lambda i: (0, 0)),
            pl.BlockSpec((D_IN, D), lambda i: (0, 0)),
            pl.BlockSpec((NC, _BR), lambda i: (0, i)),
        ],
        out_specs=[
            pl.BlockSpec((_BR, D), lambda i: (i, 0)),
            pl.BlockSpec((1, _BR), lambda i: (0, i)),
        ],
        out_shape=[
            jax.ShapeDtypeStruct((NP, D), jnp.float32),
            jax.ShapeDtypeStruct((1, NP), jnp.float32),
        ],
    )(xp, g2, b2d, W1, degp)


def _tc2_body(a0_ref, a1_ref, hs_ref, dis_ref, bias_ref, w_ref, out_ref):
    dis = dis_ref[0, :][:, None]
    t = (a0_ref[...] + a1_ref[...] + hs_ref[...]) * dis + bias_ref[...]
    t = jnp.maximum(t, 0.0)
    h2 = jnp.dot(t, w_ref[...], preferred_element_type=jnp.float32)
    out_ref[...] = h2 * dis


def _tc2(a0, a1, hs1, dis, bias, W2):
    return pl.pallas_call(
        _tc2_body,
        grid=(NP // _BR,),
        in_specs=[
            pl.BlockSpec((_BR, D), lambda i: (i, 0)),
            pl.BlockSpec((_BR, D), lambda i: (i, 0)),
            pl.BlockSpec((_BR, D), lambda i: (i, 0)),
            pl.BlockSpec((1, _BR), lambda i: (0, i)),
            pl.BlockSpec((1, D), lambda i: (0, 0)),
            pl.BlockSpec((D, D), lambda i: (0, 0)),
        ],
        out_specs=pl.BlockSpec((_BR, D), lambda i: (i, 0)),
        out_shape=jax.ShapeDtypeStruct((NP, D), jnp.float32),
    )(a0, a1, hs1, dis, bias, W2)


def _tc3_body(a0_ref, a1_ref, hs_ref, dis_ref, bias_ref, out_ref):
    dis = dis_ref[0, :][:, None]
    t = (a0_ref[...] + a1_ref[...] + hs_ref[...]) * dis + bias_ref[...]
    out_ref[...] = jnp.maximum(t, 0.0)


def _tc3(a0, a1, hs2, dis, bias):
    return pl.pallas_call(
        _tc3_body,
        grid=(NP // _BR,),
        in_specs=[
            pl.BlockSpec((_BR, D), lambda i: (i, 0)),
            pl.BlockSpec((_BR, D), lambda i: (i, 0)),
            pl.BlockSpec((_BR, D), lambda i: (i, 0)),
            pl.BlockSpec((1, _BR), lambda i: (0, i)),
            pl.BlockSpec((1, D), lambda i: (0, 0)),
        ],
        out_specs=pl.BlockSpec((_BR, D), lambda i: (i, 0)),
        out_shape=jax.ShapeDtypeStruct((NP, D), jnp.float32),
    )(a0, a1, hs2, dis, bias)


def kernel(x, edge_index, ln_gamma, ln_beta, W1, b1, W2, b2):
    src = edge_index[0].astype(jnp.int32)
    dst = edge_index[1].astype(jnp.int32)
    pad_idx = jnp.full((EP - E,), N, jnp.int32)
    srcp = jnp.concatenate([src, pad_idx]).reshape(NW * NCHUNK, CHUNK)
    dstp = jnp.concatenate([dst, pad_idx]).reshape(NW * NCHUNK, CHUNK)
    xp = jnp.pad(x, ((0, NP - N), (0, 0)))
    g2 = ln_gamma.reshape(1, D_IN)
    be2 = ln_beta.reshape(1, D_IN)
    b1r = b1.reshape(1, D)
    b2r = b2.reshape(1, D)

    degp = _deg_kernel(dstp).reshape(NC, NP)
    hs1, dis = _tc1(xp, g2, be2, W1, degp)
    a1 = _agg_kernel(hs1, srcp, dstp)
    hs2 = _tc2(a1[:NP], a1[NP:], hs1, dis, b1r, W2)
    a2 = _agg_kernel(hs2, srcp, dstp)
    out = _tc3(a2[:NP], a2[NP:], hs2, dis, b2r)
    return out[:N]


# speedup vs baseline: 38.0762x; 1.0095x over previous
import functools

import jax
import jax.numpy as jnp
from jax import lax
from jax.experimental import pallas as pl
from jax.experimental.pallas import tpu as pltpu
from jax.experimental.pallas import tpu_sc as plsc

N = 10000
E = 320000
D_IN = 128
D = 16

NC = 2
NS = 16
NW = NC * NS

NP = 10240
ROWS_PER_TILE = NP // NS

CHUNK = 128
EDGES_PER_TILE = 10240
EP = EDGES_PER_TILE * NW
NCHUNK = EDGES_PER_TILE // CHUNK
K = 8
NWAVES = NCHUNK // K

_SC_MESH = plsc.VectorSubcoreMesh(
    core_axis_name="c", subcore_axis_name="s", num_cores=NC, num_subcores=NS
)


def _tile_ids():
    cid = lax.axis_index("c")
    sid = lax.axis_index("s")
    return cid, sid


_DEG_K = 16


@functools.partial(
    pl.kernel,
    out_type=jax.ShapeDtypeStruct((NC * NP,), jnp.float32),
    mesh=_SC_MESH,
    scratch_types=[
        pltpu.VMEM((CHUNK,), jnp.float32),
        pltpu.VMEM((NCHUNK, CHUNK), jnp.int32),
        pltpu.VMEM_SHARED((NP,), jnp.float32),
        pltpu.SemaphoreType.DMA,
    ],
)
def _deg_kernel(dst_hbm, out_hbm, ones_v, idx_all, deg_sh, ssem):
    cid, sid = _tile_ids()
    tile = cid * NS + sid
    pltpu.sync_copy(dst_hbm.at[pl.ds(tile * NCHUNK, NCHUNK)], idx_all)

    def zfill(i, _):
        ones_v[pl.ds(i * 16, 16)] = jnp.zeros((16,), jnp.float32)
        return ()
    lax.fori_loop(0, CHUNK // 16, zfill, ())
    for k in range(ROWS_PER_TILE // CHUNK):
        pltpu.sync_copy(ones_v, deg_sh.at[pl.ds(sid * ROWS_PER_TILE + k * CHUNK, CHUNK)])

    def fill(i, _):
        ones_v[pl.ds(i * 16, 16)] = jnp.ones((16,), jnp.float32)
        return ()
    lax.fori_loop(0, CHUNK // 16, fill, ())
    plsc.subcore_barrier()

    nwaves = NCHUNK // _DEG_K

    def wave(w, _):
        @pl.when(w < nwaves)
        def _fire():
            for b in range(_DEG_K):
                pltpu.async_copy(ones_v, deg_sh.at[idx_all.at[w * _DEG_K + b]],
                                 ssem, add=True)

        @pl.when(w > 0)
        def _drain():
            for b in range(_DEG_K):
                pltpu.make_async_copy(ones_v, deg_sh.at[pl.ds(0, CHUNK)], ssem).wait()
        return ()
    lax.fori_loop(0, nwaves + 1, wave, ())
    plsc.subcore_barrier()

    pltpu.sync_copy(
        deg_sh.at[pl.ds(sid * ROWS_PER_TILE, ROWS_PER_TILE)],
        out_hbm.at[pl.ds(cid * NP + sid * ROWS_PER_TILE, ROWS_PER_TILE)],
    )


@functools.partial(
    pl.kernel,
    out_type=jax.ShapeDtypeStruct((NC * NP, D), jnp.float32),
    mesh=_SC_MESH,
    scratch_types=[
        pltpu.VMEM((NCHUNK, CHUNK), jnp.int32),
        pltpu.VMEM((NCHUNK, CHUNK), jnp.int32),
        pltpu.VMEM((2, K, CHUNK, D), jnp.float32),
        pltpu.VMEM_SHARED((NP, D), jnp.float32),
        pltpu.SemaphoreType.DMA,
        pltpu.SemaphoreType.DMA,
    ],
    compiler_params=pltpu.CompilerParams(use_tc_tiling_on_sc=False),
)
def _agg_kernel(table_hbm, src_hbm, dst_hbm, out_hbm,
                idx_s, idx_d, rows_v, acc_sh, gsem, ssem):
    cid, sid = _tile_ids()
    tile = cid * NS + sid
    pltpu.sync_copy(src_hbm.at[pl.ds(tile * NCHUNK, NCHUNK)], idx_s)
    pltpu.sync_copy(dst_hbm.at[pl.ds(tile * NCHUNK, NCHUNK)], idx_d)

    def zfill(i, _):
        rows_v[0, 0, i, :] = jnp.zeros((D,), jnp.float32)
        return ()
    lax.fori_loop(0, CHUNK, zfill, ())
    for k in range(ROWS_PER_TILE // CHUNK):
        pltpu.sync_copy(rows_v.at[0, 0],
                        acc_sh.at[pl.ds(sid * ROWS_PER_TILE + k * CHUNK, CHUNK)])
    plsc.subcore_barrier()

    for b in range(K):
        pltpu.async_copy(table_hbm.at[idx_s.at[b]], rows_v.at[0, b], gsem)

    def wave(w, _):
        p = lax.rem(w, 2)
        c0 = w * K
        for b in range(K):
            pltpu.make_async_copy(table_hbm.at[pl.ds(0, CHUNK)],
                                  rows_v.at[p, b], gsem).wait()
        for b in range(K):
            pltpu.async_copy(rows_v.at[p, b], acc_sh.at[idx_d.at[c0 + b]],
                             ssem, add=True)

        @pl.when(w + 1 < NWAVES)
        def _fire_next():
            for b in range(K):
                pltpu.async_copy(table_hbm.at[idx_s.at[c0 + K + b]],
                                 rows_v.at[1 - p, b], gsem)

        for b in range(K):
            pltpu.make_async_copy(rows_v.at[p, b],
                                  acc_sh.at[pl.ds(0, CHUNK)], ssem).wait()
        return ()
    lax.fori_loop(0, NWAVES, wave, ())
    plsc.subcore_barrier()

    pltpu.sync_copy(
        acc_sh.at[pl.ds(sid * ROWS_PER_TILE, ROWS_PER_TILE)],
        out_hbm.at[pl.ds(cid * NP + sid * ROWS_PER_TILE, ROWS_PER_TILE)],
    )


_BR = 1024


def _tc1_body(x_ref, g_ref, b_ref, w_ref, degp_ref, hs_ref, dis_ref):
    xb = x_ref[...]
    mean = jnp.mean(xb, axis=1, keepdims=True)
    xc = xb - mean
    var = jnp.mean(xc * xc, axis=1, keepdims=True)
    xn = xc * lax.rsqrt(var + 1e-5) * g_ref[...] + b_ref[...]
    deg = degp_ref[0, :] + degp_ref[1, :] + 1.0
    dis = lax.rsqrt(deg)
    h = jnp.dot(xn, w_ref[...], preferred_element_type=jnp.float32)
    hs_ref[...] = h * dis[:, None]
    dis_ref[...] = dis[None, :]


def _tc1(xp, g2, b2d, W1, degp):
    return pl.pallas_call(
        _tc1_body,
        grid=(NP // _BR,),
        in_specs=[
            pl.BlockSpec((_BR, D_IN), lambda i: (i, 0)),
            pl.BlockSpec((1, D_IN), lambda i: (0, 0)),
            pl.BlockSpec((1, D_IN), lambda i: (0, 0)),
            pl.BlockSpec((D_IN, D), lambda i: (0, 0)),
            pl.BlockSpec((NC, _BR), lambda i: (0, i)),
        ],
        out_specs=[
            pl.BlockSpec((_BR, D), lambda i: (i, 0)),
            pl.BlockSpec((1, _BR), lambda i: (0, i)),
        ],
        out_shape=[
            jax.ShapeDtypeStruct((NP, D), jnp.float32),
            jax.ShapeDtypeStruct((1, NP), jnp.float32),
        ],
    )(xp, g2, b2d, W1, degp)


def _tc2_body(a0_ref, a1_ref, hs_ref, dis_ref, bias_ref, w_ref, out_ref):
    dis = dis_ref[0, :][:, None]
    t = (a0_ref[...] + a1_ref[...] + hs_ref[...]) * dis + bias_ref[...]
    t = jnp.maximum(t, 0.0)
    h2 = jnp.dot(t, w_ref[...], preferred_element_type=jnp.float32)
    out_ref[...] = h2 * dis


def _tc2(a, hs1, dis, bias, W2):
    return pl.pallas_call(
        _tc2_body,
        grid=(NP // _BR,),
        in_specs=[
            pl.BlockSpec((_BR, D), lambda i: (i, 0)),
            pl.BlockSpec((_BR, D), lambda i: (i + NP // _BR, 0)),
            pl.BlockSpec((_BR, D), lambda i: (i, 0)),
            pl.BlockSpec((1, _BR), lambda i: (0, i)),
            pl.BlockSpec((1, D), lambda i: (0, 0)),
            pl.BlockSpec((D, D), lambda i: (0, 0)),
        ],
        out_specs=pl.BlockSpec((_BR, D), lambda i: (i, 0)),
        out_shape=jax.ShapeDtypeStruct((NP, D), jnp.float32),
    )(a, a, hs1, dis, bias, W2)


def _tc3_body(a0_ref, a1_ref, hs_ref, dis_ref, bias_ref, out_ref):
    dis = dis_ref[0, :][:, None]
    t = (a0_ref[...] + a1_ref[...] + hs_ref[...]) * dis + bias_ref[...]
    out_ref[...] = jnp.maximum(t, 0.0)


def _tc3(a, hs2, dis, bias):
    return pl.pallas_call(
        _tc3_body,
        grid=(NP // _BR,),
        in_specs=[
            pl.BlockSpec((_BR, D), lambda i: (i, 0)),
            pl.BlockSpec((_BR, D), lambda i: (i + NP // _BR, 0)),
            pl.BlockSpec((_BR, D), lambda i: (i, 0)),
            pl.BlockSpec((1, _BR), lambda i: (0, i)),
            pl.BlockSpec((1, D), lambda i: (0, 0)),
        ],
        out_specs=pl.BlockSpec((_BR, D), lambda i: (i, 0)),
        out_shape=jax.ShapeDtypeStruct((N, D), jnp.float32),
    )(a, a, hs2, dis, bias)


def kernel(x, edge_index, ln_gamma, ln_beta, W1, b1, W2, b2):
    src = edge_index[0].astype(jnp.int32)
    dst = edge_index[1].astype(jnp.int32)
    pad2d = jnp.full((NW * NCHUNK - E // CHUNK, CHUNK), N, jnp.int32)
    srcp = jnp.concatenate([src.reshape(E // CHUNK, CHUNK), pad2d])
    dstp = jnp.concatenate([dst.reshape(E // CHUNK, CHUNK), pad2d])
    g2 = ln_gamma.reshape(1, D_IN)
    be2 = ln_beta.reshape(1, D_IN)
    b1r = b1.reshape(1, D)
    b2r = b2.reshape(1, D)

    degp = _deg_kernel(dstp).reshape(NC, NP)
    hs1, dis = _tc1(x, g2, be2, W1, degp)
    a1 = _agg_kernel(hs1, srcp, dstp)
    hs2 = _tc2(a1, hs1, dis, b1r, W2)
    a2 = _agg_kernel(hs2, srcp, dstp)
    return _tc3(a2, hs2, dis, b2r)
